# baseline (speedup 1.0000x reference)
import functools

import jax
import jax.numpy as jnp
from jax import lax
from jax.experimental import pallas as pl
from jax.experimental.pallas import tpu as pltpu
from jax.experimental.pallas import tpu_sc as plsc

_NC = 2
_NS = 16
_NW = _NC * _NS
_L = 16

_N = 16384 * 100
_PER_W = _N // _NW


def _sc_body(x_hbm, w_hbm, out_hbm, x_v, o_v, w_v):
    wid = lax.axis_index("s") * _NC + lax.axis_index("c")
    base = wid * _PER_W

    pltpu.sync_copy(x_hbm.at[pl.ds(base, _PER_W)], x_v)
    pltpu.sync_copy(w_hbm, w_v)

    w = w_v[...]
    e = jnp.exp(w + w)
    lut = (e - 1.0) / (e + 1.0)

    def step(i, carry):
        off = i * _L
        idx = x_v[pl.ds(off, _L)]
        o_v[pl.ds(off, _L)] = lut.at[idx].get(mode="promise_in_bounds")
        return carry

    lax.fori_loop(0, _PER_W // _L, step, 0, unroll=8)

    pltpu.sync_copy(o_v, out_hbm.at[pl.ds(base, _PER_W)])


@functools.partial(jax.jit, static_argnames=())
def _run(x_flat, w_pad):
    mesh = plsc.VectorSubcoreMesh(core_axis_name="c", subcore_axis_name="s")
    f = pl.kernel(
        _sc_body,
        out_type=jax.ShapeDtypeStruct((_N,), jnp.float32),
        mesh=mesh,
        scratch_types=[
            pltpu.VMEM((_PER_W,), jnp.int32),
            pltpu.VMEM((_PER_W,), jnp.float32),
            pltpu.VMEM((_L,), jnp.float32),
        ],
    )
    return f(x_flat, w_pad)


def kernel(x, data_bias_weight):
    x_flat = x.reshape(_N)
    w_pad = jnp.pad(data_bias_weight.reshape(3), (0, _L - 3))
    out = _run(x_flat, w_pad)
    return out.reshape(16384, 100, 1)


# speedup vs baseline: 1.4540x; 1.4540x over previous
import functools

import jax
import jax.numpy as jnp
from jax import lax
from jax.experimental import pallas as pl
from jax.experimental.pallas import tpu as pltpu
from jax.experimental.pallas import tpu_sc as plsc

_NC = 2
_NS = 16
_NW = _NC * _NS
_L = 16

_ROWS = 16384
_COLS = 100
_ROWS_W = _ROWS // _NW
_CHUNK = 256
_NCHUNK = _ROWS_W // _CHUNK
_COFFS = (0, 16, 32, 48, 64, 80, 84)


def _sc_body(x_hbm, w_hbm, out_hbm, x_v, o_v, w_v):
    wid = lax.axis_index("s") * _NC + lax.axis_index("c")
    row0 = wid * _ROWS_W

    pltpu.sync_copy(w_hbm, w_v)
    w = w_v[...]
    e = jnp.exp(w + w)
    lut = (e - 1.0) / (e + 1.0)

    def row_step(r, carry):
        for c in _COFFS:
            idx = x_v[r, pl.ds(c, _L)]
            o_v[r, pl.ds(c, _L)] = lut.at[idx].get(mode="promise_in_bounds")
        return carry

    def chunk(k, carry):
        base = row0 + k * _CHUNK
        pltpu.sync_copy(x_hbm.at[pl.ds(base, _CHUNK), :], x_v)
        lax.fori_loop(0, _CHUNK, row_step, 0, unroll=4)
        pltpu.sync_copy(o_v, out_hbm.at[pl.ds(base, _CHUNK), :])
        return carry

    lax.fori_loop(0, _NCHUNK, chunk, 0)


@functools.partial(jax.jit, static_argnames=())
def _run(x, w_pad):
    mesh = plsc.VectorSubcoreMesh(core_axis_name="c", subcore_axis_name="s")
    f = pl.kernel(
        _sc_body,
        out_type=jax.ShapeDtypeStruct((_ROWS, _COLS), jnp.float32),
        mesh=mesh,
        scratch_types=[
            pltpu.VMEM((_CHUNK, _COLS), jnp.int32),
            pltpu.VMEM((_CHUNK, _COLS), jnp.float32),
            pltpu.VMEM((_L,), jnp.float32),
        ],
        compiler_params=pltpu.CompilerParams(use_tc_tiling_on_sc=True),
    )
    return f(x, w_pad)


def kernel(x, data_bias_weight):
    w_pad = jnp.pad(data_bias_weight.reshape(3), (0, _L - 3))
    return _run(x, w_pad).reshape(_ROWS, _COLS, 1)
